# Initial kernel scaffold; baseline (speedup 1.0000x reference)
import jax
import jax.numpy as jnp
from jax.experimental import pallas as pl


def kernel(points, q_coords, support_idxs, mc_points, angles, lambdas, cy_radius, cone_radius, disk_radius, disk_width, ellip_radii):
    raise NotImplementedError("write your pallas kernel here")


# speedup vs baseline: 2.5871x; 2.5871x over previous
import functools
import math

import jax
import jax.numpy as jnp
from jax import lax
from jax.experimental import pallas as pl
from jax.experimental.pallas import tpu as pltpu
from jax.experimental.pallas import tpu_sc as plsc

_REACH = 2.0
_EPS = 1e-6
_G = 16
_BM = 256
_CHUNK = 128
_D = 16


def _rotmats_host(theta):
    cx, sx = jnp.cos(theta[:, 0]), jnp.sin(theta[:, 0])
    cy, sy = jnp.cos(theta[:, 1]), jnp.sin(theta[:, 1])
    cz, sz = jnp.cos(theta[:, 2]), jnp.sin(theta[:, 2])
    one = jnp.ones_like(cx)
    zero = jnp.zeros_like(cx)
    Rx = jnp.stack([jnp.stack([one, zero, zero], -1),
                    jnp.stack([zero, cx, -sx], -1),
                    jnp.stack([zero, sx, cx], -1)], -2)
    Ry = jnp.stack([jnp.stack([cy, zero, sy], -1),
                    jnp.stack([zero, one, zero], -1),
                    jnp.stack([-sy, zero, cy], -1)], -2)
    Rz = jnp.stack([jnp.stack([cz, -sz, zero], -1),
                    jnp.stack([sz, cz, zero], -1),
                    jnp.stack([zero, zero, one], -1)], -2)
    return jnp.einsum('gij,gjk,gkl->gil', Rz, Ry, Rx)


def _make_sc_gather(n_rows, total, num_workers):
    per_w = total // num_workers
    n_vec = per_w // 16
    mesh = plsc.VectorSubcoreMesh(core_axis_name="c", subcore_axis_name="s")

    @functools.partial(
        pl.kernel, mesh=mesh,
        compiler_params=pltpu.CompilerParams(needs_layout_passes=False),
        out_type=jax.ShapeDtypeStruct((3 * total,), jnp.float32),
        scratch_types=[
            pltpu.VMEM((per_w,), jnp.int32),
            pltpu.VMEM((n_rows,), jnp.float32),
            pltpu.VMEM((per_w,), jnp.float32),
        ],
    )
    def sc_gather(idx_hbm, table_hbm, out_hbm, idx_v, tab_v, out_v):
        wid = lax.axis_index("s") * 2 + lax.axis_index("c")
        base = wid * per_w
        pltpu.sync_copy(idx_hbm.at[pl.ds(base, per_w)], idx_v)
        for c in range(3):
            pltpu.sync_copy(table_hbm.at[pl.ds(c * n_rows, n_rows)], tab_v)

            def body(i, carry):
                idx = idx_v[pl.ds(i * 16, 16)]
                out_v[pl.ds(i * 16, 16)] = plsc.load_gather(tab_v, [idx])
                return carry

            lax.fori_loop(0, n_vec, body, 0)
            pltpu.sync_copy(out_v, out_hbm.at[pl.ds(c * total + base, per_w)])

    return sc_gather


def _eval_fam(px, py, pz, g, prm_ref):
    fam, j = g // 4, g % 4
    r2 = px * px + py * py
    if fam == 0:
        r = prm_ref[0, j]
        return jnp.exp(-r2 / (2.0 * r * r + _EPS))
    if fam == 1:
        r = prm_ref[1, j]
        rz = jnp.abs(r) * jnp.clip((pz + _REACH) / (2.0 * _REACH), 0.05, 1.0)
        return jnp.exp(-r2 / (2.0 * rz * rz + _EPS))
    if fam == 2:
        r = prm_ref[2, j]
        w = prm_ref[3, j]
        return (jnp.exp(-r2 / (2.0 * r * r + _EPS))
                * jnp.exp(-pz * pz / (2.0 * w * w + _EPS)))
    ex = prm_ref[4, j]
    ey = prm_ref[5, j]
    ez = prm_ref[6, j]
    return jnp.exp(-0.5 * (px * px / (ex * ex + _EPS)
                           + py * py / (ey * ey + _EPS)
                           + pz * pz / (ez * ez + _EPS)))


def _tc_body(sup_ref, q_ref, mc_ref, r9_ref, prm_ref, lam_ref, out_ref):
    x = sup_ref[0]
    y = sup_ref[1]
    z = sup_ref[2]
    qx = q_ref[0:1, :]
    qy = q_ref[1:2, :]
    qz = q_ref[2:3, :]
    xc = x - qx
    yc = y - qy
    zc = z - qz

    mcx = mc_ref[0]
    mcy = mc_ref[1]
    mcz = mc_ref[2]
    inv_k = 1.0 / 16.0
    inv_p = 1.0 / mcx.shape[-1]

    rows = []
    for g in range(_G):
        px = r9_ref[g, 0] * xc + r9_ref[g, 1] * yc + r9_ref[g, 2] * zc
        py = r9_ref[g, 3] * xc + r9_ref[g, 4] * yc + r9_ref[g, 5] * zc
        pz = r9_ref[g, 6] * xc + r9_ref[g, 7] * yc + r9_ref[g, 8] * zc
        w = _eval_fam(px, py, pz, g, prm_ref)
        qg = jnp.sum(w, axis=0, keepdims=True) * inv_k
        wmc = _eval_fam(mcx[g:g + 1, :], mcy[g:g + 1, :], mcz[g:g + 1, :],
                        g, prm_ref)
        integ = jnp.sum(wmc) * inv_p
        rows.append(qg - integ)
    q_gib = jnp.concatenate(rows, axis=0)

    l = lam_ref[...]
    lmax = jnp.max(l, axis=0, keepdims=True)
    le = jnp.exp(l - lmax)
    lam = le / jnp.sum(le, axis=0, keepdims=True)
    out_ref[...] = lax.dot_general(
        q_gib, lam, (((0,), (0,)), ((), ())),
        preferred_element_type=jnp.float32)


def kernel(points, q_coords, support_idxs, mc_points, angles, lambdas,
           cy_radius, cone_radius, disk_radius, disk_width, ellip_radii):
    M, K = support_idxs.shape
    N = points.shape[0]
    O = lambdas.shape[1]
    num_workers = 32

    Mp = ((M + _BM - 1) // _BM) * _BM
    total = Mp * K
    assert total % (num_workers * _CHUNK) == 0

    idx = support_idxs.astype(jnp.int32)
    idx = jnp.pad(idx, ((0, Mp - M), (0, 0)))
    idx_flat = idx.reshape(total)

    table = points.T.reshape(3 * N)

    gathered = _make_sc_gather(N, total, num_workers)(idx_flat, table)

    supT = jnp.transpose(gathered.reshape(3, Mp, K), (0, 2, 1))
    qT = jnp.pad(q_coords, ((0, Mp - M), (0, 0))).T
    mcT = jnp.transpose(mc_points, (2, 0, 1))

    a = jnp.fmod(angles, 2.0)
    a = 2.0 - jax.nn.relu(-a)
    r9 = _rotmats_host(a * jnp.pi).reshape(_G, 9)

    prm = jnp.stack([
        jnp.pad(cy_radius, (0, 12)),
        jnp.pad(cone_radius, (0, 12)),
        jnp.pad(disk_radius, (0, 12)),
        jnp.pad(disk_width, (0, 12)),
        jnp.pad(ellip_radii[:, 0], (0, 12)),
        jnp.pad(ellip_radii[:, 1], (0, 12)),
        jnp.pad(ellip_radii[:, 2], (0, 12)),
        jnp.zeros((16,), jnp.float32),
    ])

    nb = Mp // _BM
    P = mc_points.shape[1]
    out = pl.pallas_call(
        _tc_body,
        grid=(nb,),
        in_specs=[
            pl.BlockSpec((3, K, _BM), lambda i: (0, 0, i)),
            pl.BlockSpec((3, _BM), lambda i: (0, i)),
            pl.BlockSpec((3, _G, P), lambda i: (0, 0, 0)),
            pl.BlockSpec((_G, 9), lambda i: (0, 0)),
            pl.BlockSpec((8, 16), lambda i: (0, 0)),
            pl.BlockSpec((_G, O), lambda i: (0, 0)),
        ],
        out_specs=pl.BlockSpec((_BM, O), lambda i: (i, 0)),
        out_shape=jax.ShapeDtypeStruct((Mp, O), jnp.float32),
    )(supT, qT, mcT, r9, prm, lambdas.astype(jnp.float32))

    return out[:M]


# speedup vs baseline: 2.9997x; 1.1595x over previous
import functools
import math

import jax
import jax.numpy as jnp
from jax import lax
from jax.experimental import pallas as pl
from jax.experimental.pallas import tpu as pltpu
from jax.experimental.pallas import tpu_sc as plsc

_REACH = 2.0
_EPS = 1e-6
_G = 16
_BM = 256
_CHUNK = 128
_D = 16


def _rotmats_host(theta):
    cx, sx = jnp.cos(theta[:, 0]), jnp.sin(theta[:, 0])
    cy, sy = jnp.cos(theta[:, 1]), jnp.sin(theta[:, 1])
    cz, sz = jnp.cos(theta[:, 2]), jnp.sin(theta[:, 2])
    one = jnp.ones_like(cx)
    zero = jnp.zeros_like(cx)
    Rx = jnp.stack([jnp.stack([one, zero, zero], -1),
                    jnp.stack([zero, cx, -sx], -1),
                    jnp.stack([zero, sx, cx], -1)], -2)
    Ry = jnp.stack([jnp.stack([cy, zero, sy], -1),
                    jnp.stack([zero, one, zero], -1),
                    jnp.stack([-sy, zero, cy], -1)], -2)
    Rz = jnp.stack([jnp.stack([cz, -sz, zero], -1),
                    jnp.stack([sz, cz, zero], -1),
                    jnp.stack([zero, zero, one], -1)], -2)
    return jnp.einsum('gij,gjk,gkl->gil', Rz, Ry, Rx)


def _make_sc_gather(n_rows, total, num_workers):
    per_w = total // num_workers
    n_vec = per_w // 16
    mesh = plsc.VectorSubcoreMesh(core_axis_name="c", subcore_axis_name="s")

    @functools.partial(
        pl.kernel, mesh=mesh,
        compiler_params=pltpu.CompilerParams(needs_layout_passes=False),
        out_type=jax.ShapeDtypeStruct((3 * total,), jnp.float32),
        scratch_types=[
            pltpu.VMEM((per_w,), jnp.int32),
            pltpu.VMEM((n_rows,), jnp.float32),
            pltpu.VMEM((per_w,), jnp.float32),
        ],
    )
    def sc_gather(idx_hbm, table_hbm, out_hbm, idx_v, tab_v, out_v):
        wid = lax.axis_index("s") * 2 + lax.axis_index("c")
        base = wid * per_w
        pltpu.sync_copy(idx_hbm.at[pl.ds(base, per_w)], idx_v)
        for c in range(3):
            pltpu.sync_copy(table_hbm.at[pl.ds(c * n_rows, n_rows)], tab_v)

            def body(i, carry):
                idx = idx_v[pl.ds(i * 16, 16)]
                out_v[pl.ds(i * 16, 16)] = plsc.load_gather(tab_v, [idx])
                return carry

            lax.fori_loop(0, n_vec, body, 0)
            pltpu.sync_copy(out_v, out_hbm.at[pl.ds(c * total + base, per_w)])

    return sc_gather


def _eval_fam(px, py, pz, g, prm_ref):
    fam, j = g // 4, g % 4
    r2 = px * px + py * py
    if fam == 0:
        r = prm_ref[0, j]
        return jnp.exp(-r2 / (2.0 * r * r + _EPS))
    if fam == 1:
        r = prm_ref[1, j]
        rz = jnp.abs(r) * jnp.clip((pz + _REACH) / (2.0 * _REACH), 0.05, 1.0)
        return jnp.exp(-r2 / (2.0 * rz * rz + _EPS))
    if fam == 2:
        r = prm_ref[2, j]
        w = prm_ref[3, j]
        return (jnp.exp(-r2 / (2.0 * r * r + _EPS))
                * jnp.exp(-pz * pz / (2.0 * w * w + _EPS)))
    ex = prm_ref[4, j]
    ey = prm_ref[5, j]
    ez = prm_ref[6, j]
    return jnp.exp(-0.5 * (px * px / (ex * ex + _EPS)
                           + py * py / (ey * ey + _EPS)
                           + pz * pz / (ez * ez + _EPS)))


def _tc_body(sup_ref, q_ref, mc_ref, r9_ref, prm_ref, lam_ref, out_ref):
    x = sup_ref[0]
    y = sup_ref[1]
    z = sup_ref[2]
    qx = q_ref[0:1, :]
    qy = q_ref[1:2, :]
    qz = q_ref[2:3, :]
    xc = x - qx
    yc = y - qy
    zc = z - qz

    mcx = mc_ref[0]
    mcy = mc_ref[1]
    mcz = mc_ref[2]
    inv_k = 1.0 / 16.0
    inv_p = 1.0 / mcx.shape[-1]

    rows = []
    for g in range(_G):
        px = r9_ref[g, 0] * xc + r9_ref[g, 1] * yc + r9_ref[g, 2] * zc
        py = r9_ref[g, 3] * xc + r9_ref[g, 4] * yc + r9_ref[g, 5] * zc
        pz = r9_ref[g, 6] * xc + r9_ref[g, 7] * yc + r9_ref[g, 8] * zc
        w = _eval_fam(px, py, pz, g, prm_ref)
        qg = jnp.sum(w, axis=0, keepdims=True) * inv_k
        wmc = _eval_fam(mcx[g:g + 1, :], mcy[g:g + 1, :], mcz[g:g + 1, :],
                        g, prm_ref)
        integ = jnp.sum(wmc) * inv_p
        rows.append(qg - integ)
    q_gib = jnp.concatenate(rows, axis=0)

    l = lam_ref[...]
    lmax = jnp.max(l, axis=0, keepdims=True)
    le = jnp.exp(l - lmax)
    lam = le / jnp.sum(le, axis=0, keepdims=True)
    out_ref[...] = lax.dot_general(
        q_gib, lam, (((0,), (0,)), ((), ())),
        preferred_element_type=jnp.float32)


def kernel(points, q_coords, support_idxs, mc_points, angles, lambdas,
           cy_radius, cone_radius, disk_radius, disk_width, ellip_radii):
    M, K = support_idxs.shape
    N = points.shape[0]
    O = lambdas.shape[1]
    num_workers = 32

    Mp = ((M + _BM - 1) // _BM) * _BM
    total = Mp * K
    assert total % (num_workers * _CHUNK) == 0

    idx = support_idxs.astype(jnp.int32)
    idx = jnp.pad(idx, ((0, Mp - M), (0, 0)))
    idx_flat = idx.T.reshape(total)

    table = points.T.reshape(3 * N)

    gathered = _make_sc_gather(N, total, num_workers)(idx_flat, table)

    supT = gathered.reshape(3, K, Mp)
    qT = jnp.pad(q_coords, ((0, Mp - M), (0, 0))).T
    mcT = jnp.transpose(mc_points, (2, 0, 1))

    a = jnp.fmod(angles, 2.0)
    a = 2.0 - jax.nn.relu(-a)
    r9 = _rotmats_host(a * jnp.pi).reshape(_G, 9)

    prm = jnp.stack([
        jnp.pad(cy_radius, (0, 12)),
        jnp.pad(cone_radius, (0, 12)),
        jnp.pad(disk_radius, (0, 12)),
        jnp.pad(disk_width, (0, 12)),
        jnp.pad(ellip_radii[:, 0], (0, 12)),
        jnp.pad(ellip_radii[:, 1], (0, 12)),
        jnp.pad(ellip_radii[:, 2], (0, 12)),
        jnp.zeros((16,), jnp.float32),
    ])

    nb = Mp // _BM
    P = mc_points.shape[1]
    out = pl.pallas_call(
        _tc_body,
        grid=(nb,),
        in_specs=[
            pl.BlockSpec((3, K, _BM), lambda i: (0, 0, i)),
            pl.BlockSpec((3, _BM), lambda i: (0, i)),
            pl.BlockSpec((3, _G, P), lambda i: (0, 0, 0)),
            pl.BlockSpec((_G, 9), lambda i: (0, 0)),
            pl.BlockSpec((8, 16), lambda i: (0, 0)),
            pl.BlockSpec((_G, O), lambda i: (0, 0)),
        ],
        out_specs=pl.BlockSpec((_BM, O), lambda i: (i, 0)),
        out_shape=jax.ShapeDtypeStruct((Mp, O), jnp.float32),
    )(supT, qT, mcT, r9, prm, lambdas.astype(jnp.float32))

    return out[:M]
